# Initial kernel scaffold; baseline (speedup 1.0000x reference)
import functools

import jax
import jax.numpy as jnp
from jax import lax
from jax.experimental import pallas as pl
from jax.experimental.pallas import tpu as pltpu
from jax.experimental.pallas import tpu_sc as plsc

N = 10000
D = 128
NNZ = 320000
NC, NS, L = 2, 16, 16
NW = NC * NS
GSZ = 128
G = 80
NNZ_PAD = NW * G * GSZ
NP = 10240
ROWS_PER_TILE = NP // NS


def _sc_mesh():
    return plsc.VectorSubcoreMesh(core_axis_name="c", subcore_axis_name="s")


def _degrees(v_blk, e_blk):
    @functools.partial(
        pl.kernel,
        out_type=(
            jax.ShapeDtypeStruct((NC, NS, ROWS_PER_TILE), jnp.float32),
            jax.ShapeDtypeStruct((NC, NS, ROWS_PER_TILE), jnp.float32),
        ),
        mesh=_sc_mesh(),
        scratch_types=[
            pltpu.VMEM((G, GSZ), jnp.int32),
            pltpu.VMEM((G, GSZ), jnp.int32),
            pltpu.VMEM((GSZ,), jnp.float32),
            pltpu.VMEM((ROWS_PER_TILE,), jnp.float32),
            pltpu.VMEM_SHARED((NP,), jnp.float32),
            pltpu.VMEM_SHARED((NP,), jnp.float32),
        ],
    )
    def k(v_hbm, e_hbm, degv_hbm, dege_hbm, idx_v, idx_e, ones, zbuf,
          accv, acce):
        c = lax.axis_index("c")
        s = lax.axis_index("s")
        wid = s * NC + c

        def fill(i, _):
            ones[pl.ds(i * L, L)] = jnp.ones((L,), jnp.float32)
            return 0

        lax.fori_loop(0, GSZ // L, fill, 0)

        def zfill(i, _):
            zbuf[pl.ds(i * L, L)] = jnp.zeros((L,), jnp.float32)
            return 0

        lax.fori_loop(0, ROWS_PER_TILE // L, zfill, 0)

        base = s * ROWS_PER_TILE
        pltpu.sync_copy(zbuf, accv.at[pl.ds(base, ROWS_PER_TILE)])
        pltpu.sync_copy(zbuf, acce.at[pl.ds(base, ROWS_PER_TILE)])
        pltpu.sync_copy(v_hbm.at[wid], idx_v)
        pltpu.sync_copy(e_hbm.at[wid], idx_e)
        plsc.subcore_barrier()

        def body(g, _):
            pltpu.sync_copy(ones, accv.at[idx_v.at[g]], add=True)
            pltpu.sync_copy(ones, acce.at[idx_e.at[g]], add=True)
            return 0

        lax.fori_loop(0, G, body, 0)
        plsc.subcore_barrier()

        pltpu.sync_copy(accv.at[pl.ds(base, ROWS_PER_TILE)], degv_hbm.at[c, s])
        pltpu.sync_copy(acce.at[pl.ds(base, ROWS_PER_TILE)], dege_hbm.at[c, s])

    return k(v_blk, e_blk)


def _smooth_phase(table, g_blk, s_blk):
    @functools.partial(
        pl.kernel,
        out_type=jax.ShapeDtypeStruct((NC, NS, ROWS_PER_TILE, D), jnp.float32),
        mesh=_sc_mesh(),
        scratch_types=[
            pltpu.VMEM((G, GSZ), jnp.int32),
            pltpu.VMEM((G, GSZ), jnp.int32),
            pltpu.VMEM((GSZ, D), jnp.float32),
            pltpu.VMEM((GSZ, D), jnp.float32),
            pltpu.VMEM_SHARED((NP, D), jnp.float32),
            pltpu.SemaphoreType.DMA,
            pltpu.SemaphoreType.DMA,
        ],
    )
    def k(tab_hbm, g_hbm, s_hbm, out_hbm, idx_g, idx_s, buf0, buf1, acc,
          sem0, sem1):
        c = lax.axis_index("c")
        s = lax.axis_index("s")
        wid = s * NC + c
        base = s * ROWS_PER_TILE

        def zrow(i, _):
            r = i // (D // L)
            col = (i % (D // L)) * L
            buf0[r, pl.ds(col, L)] = jnp.zeros((L,), jnp.float32)
            return 0

        lax.fori_loop(0, GSZ * (D // L), zrow, 0)

        for z in range(ROWS_PER_TILE // GSZ):
            pltpu.sync_copy(buf0, acc.at[pl.ds(base + z * GSZ, GSZ)])

        pltpu.sync_copy(g_hbm.at[wid], idx_g)
        pltpu.sync_copy(s_hbm.at[wid], idx_s)
        plsc.subcore_barrier()

        pltpu.async_copy(tab_hbm.at[idx_g.at[0]], buf0, sem0)
        pltpu.async_copy(tab_hbm.at[idx_g.at[1]], buf1, sem1)

        def body(i, _):
            g0 = 2 * i
            g1 = 2 * i + 1
            pltpu.make_async_copy(tab_hbm.at[idx_g.at[g0]], buf0, sem0).wait()
            pltpu.sync_copy(buf0, acc.at[idx_s.at[g0]], add=True)

            @pl.when(g0 + 2 < G)
            def _():
                pltpu.async_copy(tab_hbm.at[idx_g.at[g0 + 2]], buf0, sem0)

            pltpu.make_async_copy(tab_hbm.at[idx_g.at[g1]], buf1, sem1).wait()
            pltpu.sync_copy(buf1, acc.at[idx_s.at[g1]], add=True)

            @pl.when(g1 + 2 < G)
            def _():
                pltpu.async_copy(tab_hbm.at[idx_g.at[g1 + 2]], buf1, sem1)

            return 0

        lax.fori_loop(0, G // 2, body, 0)
        plsc.subcore_barrier()

        pltpu.sync_copy(acc.at[pl.ds(base, ROWS_PER_TILE)], out_hbm.at[c, s])

    return k(table, g_blk, s_blk)


_BROWS = 1280


def _inv_kernel(dv_ref, de_ref, isdv_ref, ide_ref):
    dv = dv_ref[0] + dv_ref[1]
    de = de_ref[0] + de_ref[1]
    isdv_ref[...] = jnp.where(dv > 0, lax.rsqrt(dv), 0.0)
    ide_ref[...] = jnp.where(de > 0, 1.0 / de, 0.0)


def _inv_vectors(degv, dege):
    return pl.pallas_call(
        _inv_kernel,
        grid=(NP // _BROWS,),
        in_specs=[
            pl.BlockSpec((2, _BROWS, 1), lambda i: (0, i, 0)),
            pl.BlockSpec((2, _BROWS, 1), lambda i: (0, i, 0)),
        ],
        out_specs=[
            pl.BlockSpec((_BROWS, 1), lambda i: (i, 0)),
            pl.BlockSpec((_BROWS, 1), lambda i: (i, 0)),
        ],
        out_shape=[
            jax.ShapeDtypeStruct((NP, 1), jnp.float32),
            jax.ShapeDtypeStruct((NP, 1), jnp.float32),
        ],
    )(degv, dege)


def _proj_kernel(x_ref, w_ref, b_ref, s_ref, y_ref):
    y = jnp.dot(x_ref[...], w_ref[...].T, preferred_element_type=jnp.float32)
    y_ref[...] = (y + b_ref[...]) * s_ref[...]


def _project(x_pad, w, b, isdv):
    return pl.pallas_call(
        _proj_kernel,
        grid=(NP // _BROWS,),
        in_specs=[
            pl.BlockSpec((_BROWS, D), lambda i: (i, 0)),
            pl.BlockSpec((D, D), lambda i: (0, 0)),
            pl.BlockSpec((1, D), lambda i: (0, 0)),
            pl.BlockSpec((_BROWS, 1), lambda i: (i, 0)),
        ],
        out_specs=pl.BlockSpec((_BROWS, D), lambda i: (i, 0)),
        out_shape=jax.ShapeDtypeStruct((NP, D), jnp.float32),
    )(x_pad, w, b, isdv)


def _combine_kernel(p_ref, s_ref, o_ref):
    o_ref[...] = (p_ref[0] + p_ref[1]) * s_ref[...]


def _combine_scale(parts, scale):
    return pl.pallas_call(
        _combine_kernel,
        grid=(NP // _BROWS,),
        in_specs=[
            pl.BlockSpec((2, _BROWS, D), lambda i: (0, i, 0)),
            pl.BlockSpec((_BROWS, 1), lambda i: (i, 0)),
        ],
        out_specs=pl.BlockSpec((_BROWS, D), lambda i: (i, 0)),
        out_shape=jax.ShapeDtypeStruct((NP, D), jnp.float32),
    )(parts, scale)


def _final_kernel(p_ref, s_ref, o_ref):
    o_ref[...] = jnp.maximum((p_ref[0] + p_ref[1]) * s_ref[...], 0.0)


_FROWS = 2000


def _final(parts, isdv):
    return pl.pallas_call(
        _final_kernel,
        grid=(N // _FROWS,),
        in_specs=[
            pl.BlockSpec((2, _FROWS, D), lambda i: (0, i, 0)),
            pl.BlockSpec((_FROWS, 1), lambda i: (i, 0)),
        ],
        out_specs=pl.BlockSpec((_FROWS, D), lambda i: (i, 0)),
        out_shape=jax.ShapeDtypeStruct((N, D), jnp.float32),
    )(parts, isdv)


def kernel(X, v_idx, e_idx, W, b):
    pad = jnp.full((NNZ_PAD - NNZ,), N, dtype=jnp.int32)
    v_blk = jnp.concatenate([v_idx, pad]).reshape(NW, G, GSZ)
    e_blk = jnp.concatenate([e_idx, pad]).reshape(NW, G, GSZ)
    x_pad = jnp.concatenate(
        [X, jnp.zeros((NP - N, D), dtype=jnp.float32)], axis=0)

    degv, dege = _degrees(v_blk, e_blk)
    degv = degv.reshape(NC, NP, 1)
    dege = dege.reshape(NC, NP, 1)
    isdv, ide = _inv_vectors(degv, dege)

    y = _project(x_pad, W, b.reshape(1, D), isdv)

    edge_parts = _smooth_phase(y, v_blk, e_blk).reshape(NC, NP, D)
    edge_feat = _combine_scale(edge_parts, ide)

    vert_parts = _smooth_phase(edge_feat, e_blk, v_blk).reshape(NC, NP, D)
    return _final(vert_parts, isdv)


# speedup vs baseline: 5.1222x; 5.1222x over previous
import functools

import jax
import jax.numpy as jnp
from jax import lax
from jax.experimental import pallas as pl
from jax.experimental.pallas import tpu as pltpu
from jax.experimental.pallas import tpu_sc as plsc

N = 10000
D = 128
DH = D // 2
NNZ = 320000
NC, NS, L = 2, 16, 16
GSZ = 128
G = 160
NNZ_PAD = NS * G * GSZ
NP = 10240
ROWS_PER_TILE = NP // NS
GD = 80


def _sc_mesh():
    return plsc.VectorSubcoreMesh(core_axis_name="c", subcore_axis_name="s")


def _degrees(v_blk, e_blk):
    @functools.partial(
        pl.kernel,
        out_type=(
            jax.ShapeDtypeStruct((NC, NS, ROWS_PER_TILE), jnp.float32),
            jax.ShapeDtypeStruct((NC, NS, ROWS_PER_TILE), jnp.float32),
        ),
        mesh=_sc_mesh(),
        scratch_types=[
            pltpu.VMEM((GD, GSZ), jnp.int32),
            pltpu.VMEM((GD, GSZ), jnp.int32),
            pltpu.VMEM((GSZ,), jnp.float32),
            pltpu.VMEM((ROWS_PER_TILE,), jnp.float32),
            pltpu.VMEM_SHARED((NP,), jnp.float32),
            pltpu.VMEM_SHARED((NP,), jnp.float32),
        ],
    )
    def k(v_hbm, e_hbm, degv_hbm, dege_hbm, idx_v, idx_e, ones, zbuf,
          accv, acce):
        c = lax.axis_index("c")
        s = lax.axis_index("s")
        wid = s * NC + c

        def fill(i, _):
            ones[pl.ds(i * L, L)] = jnp.ones((L,), jnp.float32)
            return 0

        lax.fori_loop(0, GSZ // L, fill, 0)

        def zfill(i, _):
            zbuf[pl.ds(i * L, L)] = jnp.zeros((L,), jnp.float32)
            return 0

        lax.fori_loop(0, ROWS_PER_TILE // L, zfill, 0)

        base = s * ROWS_PER_TILE
        pltpu.sync_copy(zbuf, accv.at[pl.ds(base, ROWS_PER_TILE)])
        pltpu.sync_copy(zbuf, acce.at[pl.ds(base, ROWS_PER_TILE)])
        pltpu.sync_copy(v_hbm.at[wid], idx_v)
        pltpu.sync_copy(e_hbm.at[wid], idx_e)
        plsc.subcore_barrier()

        def body(g, _):
            pltpu.sync_copy(ones, accv.at[idx_v.at[g]], add=True)
            pltpu.sync_copy(ones, acce.at[idx_e.at[g]], add=True)
            return 0

        lax.fori_loop(0, GD, body, 0)
        plsc.subcore_barrier()

        pltpu.sync_copy(accv.at[pl.ds(base, ROWS_PER_TILE)], degv_hbm.at[c, s])
        pltpu.sync_copy(acce.at[pl.ds(base, ROWS_PER_TILE)], dege_hbm.at[c, s])

    return k(v_blk, e_blk)


def _smooth_phase(table, g_blk, s_blk):
    @functools.partial(
        pl.kernel,
        out_type=jax.ShapeDtypeStruct((NC, NS, ROWS_PER_TILE, DH),
                                      jnp.float32),
        mesh=_sc_mesh(),
        compiler_params=pltpu.CompilerParams(use_tc_tiling_on_sc=False),
        scratch_types=[
            pltpu.VMEM((G, GSZ), jnp.int32),
            pltpu.VMEM((G, GSZ), jnp.int32),
            pltpu.VMEM((GSZ, DH), jnp.float32),
            pltpu.VMEM((GSZ, DH), jnp.float32),
            pltpu.VMEM_SHARED((NP, DH), jnp.float32),
            pltpu.SemaphoreType.DMA,
            pltpu.SemaphoreType.DMA,
        ],
    )
    def k(tab_hbm, g_hbm, s_hbm, out_hbm, idx_g, idx_s, buf0, buf1, acc,
          sem0, sem1):
        c = lax.axis_index("c")
        s = lax.axis_index("s")
        base = s * ROWS_PER_TILE
        tab = tab_hbm.at[c]

        def zrow(i, _):
            r = i // (DH // L)
            col = (i % (DH // L)) * L
            buf0[r, pl.ds(col, L)] = jnp.zeros((L,), jnp.float32)
            return 0

        lax.fori_loop(0, GSZ * (DH // L), zrow, 0)

        for z in range(ROWS_PER_TILE // GSZ):
            pltpu.sync_copy(buf0, acc.at[pl.ds(base + z * GSZ, GSZ)])

        pltpu.sync_copy(g_hbm.at[s], idx_g)
        pltpu.sync_copy(s_hbm.at[s], idx_s)
        plsc.subcore_barrier()

        pltpu.async_copy(tab.at[idx_g.at[0]], buf0, sem0)
        pltpu.async_copy(tab.at[idx_g.at[1]], buf1, sem1)

        def body(i, _):
            g0 = 2 * i
            g1 = 2 * i + 1
            pltpu.make_async_copy(tab.at[idx_g.at[g0]], buf0, sem0).wait()
            pltpu.sync_copy(buf0, acc.at[idx_s.at[g0]], add=True)

            @pl.when(g0 + 2 < G)
            def _():
                pltpu.async_copy(tab.at[idx_g.at[g0 + 2]], buf0, sem0)

            pltpu.make_async_copy(tab.at[idx_g.at[g1]], buf1, sem1).wait()
            pltpu.sync_copy(buf1, acc.at[idx_s.at[g1]], add=True)

            @pl.when(g1 + 2 < G)
            def _():
                pltpu.async_copy(tab.at[idx_g.at[g1 + 2]], buf1, sem1)

            return 0

        lax.fori_loop(0, G // 2, body, 0)
        plsc.subcore_barrier()

        pltpu.sync_copy(acc.at[pl.ds(base, ROWS_PER_TILE)], out_hbm.at[c, s])

    return k(table, g_blk, s_blk)


_BROWS = 1280


def _inv_kernel(dv_ref, de_ref, isdv_ref, ide_ref):
    dv = dv_ref[0] + dv_ref[1]
    de = de_ref[0] + de_ref[1]
    isdv_ref[...] = jnp.where(dv > 0, lax.rsqrt(dv), 0.0)
    ide_ref[...] = jnp.where(de > 0, 1.0 / de, 0.0)


def _inv_vectors(degv, dege):
    return pl.pallas_call(
        _inv_kernel,
        grid=(NP // _BROWS,),
        in_specs=[
            pl.BlockSpec((2, _BROWS, 1), lambda i: (0, i, 0)),
            pl.BlockSpec((2, _BROWS, 1), lambda i: (0, i, 0)),
        ],
        out_specs=[
            pl.BlockSpec((_BROWS, 1), lambda i: (i, 0)),
            pl.BlockSpec((_BROWS, 1), lambda i: (i, 0)),
        ],
        out_shape=[
            jax.ShapeDtypeStruct((NP, 1), jnp.float32),
            jax.ShapeDtypeStruct((NP, 1), jnp.float32),
        ],
    )(degv, dege)


def _proj_kernel(x_ref, w_ref, b_ref, s_ref, y_ref):
    y = jnp.dot(x_ref[...], w_ref[...].T, preferred_element_type=jnp.float32)
    y = (y + b_ref[...]) * s_ref[...]
    y_ref[0] = y[:, :DH]
    y_ref[1] = y[:, DH:]


def _project(x_pad, w, b, isdv):
    return pl.pallas_call(
        _proj_kernel,
        grid=(NP // _BROWS,),
        in_specs=[
            pl.BlockSpec((_BROWS, D), lambda i: (i, 0)),
            pl.BlockSpec((D, D), lambda i: (0, 0)),
            pl.BlockSpec((1, D), lambda i: (0, 0)),
            pl.BlockSpec((_BROWS, 1), lambda i: (i, 0)),
        ],
        out_specs=pl.BlockSpec((2, _BROWS, DH), lambda i: (0, i, 0)),
        out_shape=jax.ShapeDtypeStruct((NC, NP, DH), jnp.float32),
    )(x_pad, w, b, isdv)


def _scale_kernel(p_ref, s_ref, o_ref):
    o_ref[...] = p_ref[...] * s_ref[...]


def _scale_split(parts, scale):
    return pl.pallas_call(
        _scale_kernel,
        grid=(NP // _BROWS,),
        in_specs=[
            pl.BlockSpec((2, _BROWS, DH), lambda i: (0, i, 0)),
            pl.BlockSpec((_BROWS, 1), lambda i: (i, 0)),
        ],
        out_specs=pl.BlockSpec((2, _BROWS, DH), lambda i: (0, i, 0)),
        out_shape=jax.ShapeDtypeStruct((NC, NP, DH), jnp.float32),
    )(parts, scale)


def _final_kernel(p_ref, s_ref, o_ref):
    y = jnp.concatenate([p_ref[0], p_ref[1]], axis=1) * s_ref[...]
    o_ref[...] = jnp.maximum(y, 0.0)


_FROWS = 2000


def _final(parts, isdv):
    return pl.pallas_call(
        _final_kernel,
        grid=(N // _FROWS,),
        in_specs=[
            pl.BlockSpec((2, _FROWS, DH), lambda i: (0, i, 0)),
            pl.BlockSpec((_FROWS, 1), lambda i: (i, 0)),
        ],
        out_specs=pl.BlockSpec((_FROWS, D), lambda i: (i, 0)),
        out_shape=jax.ShapeDtypeStruct((N, D), jnp.float32),
    )(parts, isdv)


def kernel(X, v_idx, e_idx, W, b):
    pad = jnp.full((NNZ_PAD - NNZ,), N, dtype=jnp.int32)
    v_blk = jnp.concatenate([v_idx, pad]).reshape(NS, G, GSZ)
    e_blk = jnp.concatenate([e_idx, pad]).reshape(NS, G, GSZ)
    v32 = v_blk.reshape(NC * NS, GD, GSZ)
    e32 = e_blk.reshape(NC * NS, GD, GSZ)
    x_pad = jnp.concatenate(
        [X, jnp.zeros((NP - N, D), dtype=jnp.float32)], axis=0)

    degv, dege = _degrees(v32, e32)
    degv = degv.reshape(NC, NP, 1)
    dege = dege.reshape(NC, NP, 1)
    isdv, ide = _inv_vectors(degv, dege)

    y = _project(x_pad, W, b.reshape(1, D), isdv)

    edge_sums = _smooth_phase(y, v_blk, e_blk)
    edge_feat = _scale_split(edge_sums.reshape(NC, NP, DH), ide)

    vert_sums = _smooth_phase(edge_feat, e_blk, v_blk)
    return _final(vert_sums.reshape(NC, NP, DH), isdv)


# speedup vs baseline: 5.1653x; 1.0084x over previous
import functools

import jax
import jax.numpy as jnp
from jax import lax
from jax.experimental import pallas as pl
from jax.experimental.pallas import tpu as pltpu
from jax.experimental.pallas import tpu_sc as plsc

N = 10000
D = 128
DH = D // 2
NNZ = 320000
NC, NS, L = 2, 16, 16
GSZ = 128
G = 160
NNZ_PAD = NS * G * GSZ
NP = 10240
ROWS_PER_TILE = NP // NS
GD = 80
NBUF = 4


def _sc_mesh():
    return plsc.VectorSubcoreMesh(core_axis_name="c", subcore_axis_name="s")


def _degrees(v_blk, e_blk):
    @functools.partial(
        pl.kernel,
        out_type=(
            jax.ShapeDtypeStruct((NC, NS, ROWS_PER_TILE), jnp.float32),
            jax.ShapeDtypeStruct((NC, NS, ROWS_PER_TILE), jnp.float32),
        ),
        mesh=_sc_mesh(),
        scratch_types=[
            pltpu.VMEM((GD, GSZ), jnp.int32),
            pltpu.VMEM((GD, GSZ), jnp.int32),
            pltpu.VMEM((GSZ,), jnp.float32),
            pltpu.VMEM((ROWS_PER_TILE,), jnp.float32),
            pltpu.VMEM_SHARED((NP,), jnp.float32),
            pltpu.VMEM_SHARED((NP,), jnp.float32),
        ],
    )
    def k(v_hbm, e_hbm, degv_hbm, dege_hbm, idx_v, idx_e, ones, zbuf,
          accv, acce):
        c = lax.axis_index("c")
        s = lax.axis_index("s")
        wid = s * NC + c

        def fill(i, _):
            ones[pl.ds(i * L, L)] = jnp.ones((L,), jnp.float32)
            return 0

        lax.fori_loop(0, GSZ // L, fill, 0)

        def zfill(i, _):
            zbuf[pl.ds(i * L, L)] = jnp.zeros((L,), jnp.float32)
            return 0

        lax.fori_loop(0, ROWS_PER_TILE // L, zfill, 0)

        base = s * ROWS_PER_TILE
        pltpu.sync_copy(zbuf, accv.at[pl.ds(base, ROWS_PER_TILE)])
        pltpu.sync_copy(zbuf, acce.at[pl.ds(base, ROWS_PER_TILE)])
        pltpu.sync_copy(v_hbm.at[wid], idx_v)
        pltpu.sync_copy(e_hbm.at[wid], idx_e)
        plsc.subcore_barrier()

        def body(g, _):
            pltpu.sync_copy(ones, accv.at[idx_v.at[g]], add=True)
            pltpu.sync_copy(ones, acce.at[idx_e.at[g]], add=True)
            return 0

        lax.fori_loop(0, GD, body, 0)
        plsc.subcore_barrier()

        pltpu.sync_copy(accv.at[pl.ds(base, ROWS_PER_TILE)], degv_hbm.at[c, s])
        pltpu.sync_copy(acce.at[pl.ds(base, ROWS_PER_TILE)], dege_hbm.at[c, s])

    return k(v_blk, e_blk)


def _smooth_phase(table, g_blk, s_blk):
    @functools.partial(
        pl.kernel,
        out_type=jax.ShapeDtypeStruct((NC, NS, ROWS_PER_TILE, DH),
                                      jnp.float32),
        mesh=_sc_mesh(),
        compiler_params=pltpu.CompilerParams(use_tc_tiling_on_sc=False),
        scratch_types=[
            pltpu.VMEM((G, GSZ), jnp.int32),
            pltpu.VMEM((G, GSZ), jnp.int32),
            [pltpu.VMEM((GSZ, DH), jnp.float32) for _ in range(NBUF)],
            pltpu.VMEM_SHARED((NP, DH), jnp.float32),
            [pltpu.SemaphoreType.DMA for _ in range(NBUF)],
            [pltpu.SemaphoreType.DMA for _ in range(NBUF)],
        ],
    )
    def k(tab_hbm, g_hbm, s_hbm, out_hbm, idx_g, idx_s, bufs, acc,
          gsem, ssem):
        c = lax.axis_index("c")
        s = lax.axis_index("s")
        base = s * ROWS_PER_TILE
        tab = tab_hbm.at[c]

        def zrow(i, _):
            r = i // (DH // L)
            col = (i % (DH // L)) * L
            bufs[0][r, pl.ds(col, L)] = jnp.zeros((L,), jnp.float32)
            return 0

        lax.fori_loop(0, GSZ * (DH // L), zrow, 0)

        for z in range(ROWS_PER_TILE // GSZ):
            pltpu.sync_copy(bufs[0], acc.at[pl.ds(base + z * GSZ, GSZ)])

        pltpu.sync_copy(g_hbm.at[s], idx_g)
        pltpu.sync_copy(s_hbm.at[s], idx_s)
        plsc.subcore_barrier()

        for b in range(NBUF):
            pltpu.async_copy(tab.at[idx_g.at[b]], bufs[b], gsem[b])

        def body(i, _):
            for b in range(NBUF):
                g = NBUF * i + b
                pltpu.make_async_copy(
                    tab.at[idx_g.at[g]], bufs[b], gsem[b]).wait()
                pltpu.async_copy(
                    bufs[b], acc.at[idx_s.at[g]], ssem[b], add=True)

            @pl.when(i + 1 < G // NBUF)
            def _():
                for b in range(NBUF):
                    g = NBUF * (i + 1) + b
                    pltpu.make_async_copy(
                        bufs[b], acc.at[idx_s.at[g - NBUF]], ssem[b]).wait()
                    pltpu.async_copy(tab.at[idx_g.at[g]], bufs[b], gsem[b])

            return 0

        lax.fori_loop(0, G // NBUF, body, 0)
        for b in range(NBUF):
            pltpu.make_async_copy(
                bufs[b], acc.at[idx_s.at[G - NBUF + b]], ssem[b]).wait()
        plsc.subcore_barrier()

        pltpu.sync_copy(acc.at[pl.ds(base, ROWS_PER_TILE)], out_hbm.at[c, s])

    return k(table, g_blk, s_blk)


_BROWS = 1280


def _proj_kernel(x_ref, w_ref, b_ref, dv_ref, de_ref, y_ref, isdv_ref,
                 ide_ref):
    dv = dv_ref[0] + dv_ref[1]
    de = de_ref[0] + de_ref[1]
    isdv = jnp.where(dv > 0, lax.rsqrt(dv), 0.0)
    isdv_ref[...] = isdv
    ide_ref[...] = jnp.where(de > 0, 1.0 / de, 0.0)
    y = jnp.dot(x_ref[...], w_ref[...].T, preferred_element_type=jnp.float32)
    y = (y + b_ref[...]) * isdv
    y_ref[0] = y[:, :DH]
    y_ref[1] = y[:, DH:]


def _project(x_pad, w, b, degv, dege):
    return pl.pallas_call(
        _proj_kernel,
        grid=(NP // _BROWS,),
        in_specs=[
            pl.BlockSpec((_BROWS, D), lambda i: (i, 0)),
            pl.BlockSpec((D, D), lambda i: (0, 0)),
            pl.BlockSpec((1, D), lambda i: (0, 0)),
            pl.BlockSpec((2, _BROWS, 1), lambda i: (0, i, 0)),
            pl.BlockSpec((2, _BROWS, 1), lambda i: (0, i, 0)),
        ],
        out_specs=[
            pl.BlockSpec((2, _BROWS, DH), lambda i: (0, i, 0)),
            pl.BlockSpec((_BROWS, 1), lambda i: (i, 0)),
            pl.BlockSpec((_BROWS, 1), lambda i: (i, 0)),
        ],
        out_shape=[
            jax.ShapeDtypeStruct((NC, NP, DH), jnp.float32),
            jax.ShapeDtypeStruct((NP, 1), jnp.float32),
            jax.ShapeDtypeStruct((NP, 1), jnp.float32),
        ],
    )(x_pad, w, b, degv, dege)


def _scale_kernel(p_ref, s_ref, o_ref):
    o_ref[...] = p_ref[...] * s_ref[...]


def _scale_split(parts, scale):
    return pl.pallas_call(
        _scale_kernel,
        grid=(NP // _BROWS,),
        in_specs=[
            pl.BlockSpec((2, _BROWS, DH), lambda i: (0, i, 0)),
            pl.BlockSpec((_BROWS, 1), lambda i: (i, 0)),
        ],
        out_specs=pl.BlockSpec((2, _BROWS, DH), lambda i: (0, i, 0)),
        out_shape=jax.ShapeDtypeStruct((NC, NP, DH), jnp.float32),
    )(parts, scale)


def _final_kernel(p_ref, s_ref, o_ref):
    y = jnp.concatenate([p_ref[0], p_ref[1]], axis=1) * s_ref[...]
    o_ref[...] = jnp.maximum(y, 0.0)


_FROWS = 2000


def _final(parts, isdv):
    return pl.pallas_call(
        _final_kernel,
        grid=(N // _FROWS,),
        in_specs=[
            pl.BlockSpec((2, _FROWS, DH), lambda i: (0, i, 0)),
            pl.BlockSpec((_FROWS, 1), lambda i: (i, 0)),
        ],
        out_specs=pl.BlockSpec((_FROWS, D), lambda i: (i, 0)),
        out_shape=jax.ShapeDtypeStruct((N, D), jnp.float32),
    )(parts, isdv)


def kernel(X, v_idx, e_idx, W, b):
    pad = jnp.full((NNZ_PAD - NNZ,), N, dtype=jnp.int32)
    v_blk = jnp.concatenate([v_idx, pad]).reshape(NS, G, GSZ)
    e_blk = jnp.concatenate([e_idx, pad]).reshape(NS, G, GSZ)
    v32 = v_blk.reshape(NC * NS, GD, GSZ)
    e32 = e_blk.reshape(NC * NS, GD, GSZ)
    x_pad = jnp.concatenate(
        [X, jnp.zeros((NP - N, D), dtype=jnp.float32)], axis=0)

    degv, dege = _degrees(v32, e32)
    degv = degv.reshape(NC, NP, 1)
    dege = dege.reshape(NC, NP, 1)
    y, isdv, ide = _project(x_pad, W, b.reshape(1, D), degv, dege)

    edge_sums = _smooth_phase(y, v_blk, e_blk)
    edge_feat = _scale_split(edge_sums.reshape(NC, NP, DH), ide)

    vert_sums = _smooth_phase(edge_feat, e_blk, v_blk)
    return _final(vert_sums.reshape(NC, NP, DH), isdv)
